# baseline (speedup 1.0000x reference)
import functools

import jax
import jax.numpy as jnp
from jax import lax
from jax.experimental import pallas as pl
from jax.experimental.pallas import tpu as pltpu
from jax.experimental.pallas import tpu_sc as plsc

_INFO = plsc.get_sparse_core_info()
_NC, _NS = _INFO.num_cores, _INFO.num_subcores
_NW = _NC * _NS

_G = 128
_JB = 4
_D = 64
_B = 200
_BPC = 2
_CH = _B * _BPC

_N_TRAJ = 16384 * _B
_N_USER = 16384
_BATCH_PER_W = 16384 // _NW
_TRAJ_CHUNKS = _BATCH_PER_W // _BPC
_USER_CHUNKS = 1
_UCH = _JB * _G


def _traj_body(traj_hbm, ltab_hbm, tout_hbm, idx_v, rows_v, isem, gsem, osem):
  wid = lax.axis_index("s") * _NC + lax.axis_index("c")
  bbase = wid * _BATCH_PER_W

  def start_idx(g, b):
    pltpu.async_copy(traj_hbm.at[pl.ds((bbase + g * _BPC) * _B, _CH)],
                     idx_v.at[b], isem.at[b])

  def wait_idx(b):
    pltpu.make_async_copy(traj_hbm.at[pl.ds(0, _CH)], idx_v.at[b],
                          isem.at[b]).wait()

  def start_gathers(b):
    for k in range(_BPC):
      pltpu.async_copy(ltab_hbm.at[idx_v.at[b, pl.ds(k * _B, _B)]],
                       rows_v.at[b, k], gsem.at[b])

  def wait_gathers(b):
    for k in range(_BPC):
      pltpu.make_async_copy(ltab_hbm.at[pl.ds(0, _B)], rows_v.at[b, k],
                            gsem.at[b]).wait()

  def start_out(g, b):
    pltpu.async_copy(rows_v.at[b],
                     tout_hbm.at[pl.ds(bbase + g * _BPC, _BPC)],
                     osem.at[b])

  def wait_out(b):
    pltpu.make_async_copy(rows_v.at[b], tout_hbm.at[pl.ds(0, _BPC)],
                          osem.at[b]).wait()

  start_idx(0, 0)
  start_idx(1, 1)
  start_idx(2, 2)
  wait_idx(0)
  start_gathers(0)
  wait_idx(1)
  start_gathers(1)

  @pl.loop(0, _TRAJ_CHUNKS)
  def _(g):
    b0 = lax.rem(g, 3)
    b2 = lax.rem(g + 2, 3)

    @pl.when(g >= 1)
    def _():
      wait_out(b2)

    @pl.when(g + 2 < _TRAJ_CHUNKS)
    def _():
      wait_idx(b2)
      start_gathers(b2)
    wait_gathers(b0)

    @pl.when(g + 3 < _TRAJ_CHUNKS)
    def _():
      start_idx(g + 3, b0)
    start_out(g, b0)

  wait_out((_TRAJ_CHUNKS - 1) % 3)


def _user_body(user_hbm, utab_hbm, uout_hbm, idx_v, rows_v, gsem):
  wid = lax.axis_index("s") * _NC + lax.axis_index("c")
  ubase = wid * _USER_CHUNKS
  pltpu.sync_copy(user_hbm.at[pl.ds(ubase * _JB, _JB)], idx_v)
  copies = [
      pltpu.async_copy(utab_hbm.at[idx_v.at[j]],
                       rows_v.at[pl.ds(j * _G, _G)], gsem)
      for j in range(_JB)
  ]
  for c in copies:
    c.wait()
  pltpu.sync_copy(rows_v,
                  uout_hbm.at[pl.ds(ubase * _UCH, _UCH), pl.ds(0, _D)])


@jax.jit
def _sc_embed(user2d, traj1d, user_table, loc_table):
  mesh = plsc.VectorSubcoreMesh(core_axis_name="c", subcore_axis_name="s")
  traj_fn = pl.kernel(
      _traj_body,
      out_type=jax.ShapeDtypeStruct((16384, _B, _D), jnp.float32),
      mesh=mesh,
      compiler_params=pltpu.CompilerParams(use_tc_tiling_on_sc=False),
      scratch_types=[
          pltpu.VMEM((3, _CH), jnp.int32),
          pltpu.VMEM((3, _BPC, _B, _D), jnp.float32),
          pltpu.SemaphoreType.DMA((3,)),
          pltpu.SemaphoreType.DMA((3,)),
          pltpu.SemaphoreType.DMA((3,)),
      ],
  )
  user_fn = pl.kernel(
      _user_body,
      out_type=jax.ShapeDtypeStruct((_N_USER, 2 * _D), jnp.float32),
      mesh=mesh,
      compiler_params=pltpu.CompilerParams(use_tc_tiling_on_sc=False),
      scratch_types=[
          pltpu.VMEM((_JB, _G), jnp.int32),
          pltpu.VMEM((_UCH, _D), jnp.float32),
          pltpu.SemaphoreType.DMA,
      ],
  )
  tout = traj_fn(traj1d, loc_table)
  uout = user_fn(user2d, user_table)
  return uout, tout


def kernel(user, traj, user_table, loc_table):
  user2d = user.astype(jnp.int32).reshape(_N_USER // _G, _G)
  traj1d = traj.reshape(-1)
  uout128, tout = _sc_embed(user2d, traj1d, user_table, loc_table)
  return (uout128[:, :_D], tout)


# speedup vs baseline: 1.6823x; 1.6823x over previous
import functools

import jax
import jax.numpy as jnp
from jax import lax
from jax.experimental import pallas as pl
from jax.experimental.pallas import tpu as pltpu
from jax.experimental.pallas import tpu_sc as plsc

_INFO = plsc.get_sparse_core_info()
_NC, _NS = _INFO.num_cores, _INFO.num_subcores
_NW = _NC * _NS

_G = 128
_CH = 512
_D = 64

_N_TRAJ = 16384 * 200
_N_USER = 16384
_TRAJ_CHUNKS = _N_TRAJ // (_NW * _CH)
_UCH = _N_USER // _NW


def _traj_body(traj_hbm, ltab_hbm, tout_hbm, idx_v, rows_v, isem, gsem, osem):
  wid = lax.axis_index("s") * _NC + lax.axis_index("c")
  base = wid * _TRAJ_CHUNKS

  def start_idx(g, b):
    pltpu.async_copy(traj_hbm.at[pl.ds((base + g) * _CH, _CH)],
                     idx_v.at[b], isem.at[b])

  def wait_idx(b):
    pltpu.make_async_copy(traj_hbm.at[pl.ds(0, _CH)], idx_v.at[b],
                          isem.at[b]).wait()

  def start_gathers(b):
    pltpu.async_copy(ltab_hbm.at[idx_v.at[b]], rows_v.at[b], gsem.at[b])

  def wait_gathers(b):
    pltpu.make_async_copy(ltab_hbm.at[pl.ds(0, _CH)], rows_v.at[b],
                          gsem.at[b]).wait()

  def start_out(g, b):
    pltpu.async_copy(rows_v.at[b],
                     tout_hbm.at[pl.ds((base + g) * _CH, _CH), pl.ds(0, _D)],
                     osem.at[b])

  def wait_out(b):
    pltpu.make_async_copy(rows_v.at[b],
                          tout_hbm.at[pl.ds(0, _CH), pl.ds(0, _D)],
                          osem.at[b]).wait()

  start_idx(0, 0)
  start_idx(1, 1)
  start_idx(2, 2)
  wait_idx(0)
  start_gathers(0)
  wait_idx(1)
  start_gathers(1)

  @pl.loop(0, _TRAJ_CHUNKS)
  def _(g):
    b0 = lax.rem(g, 3)
    b2 = lax.rem(g + 2, 3)

    @pl.when(g >= 1)
    def _():
      wait_out(b2)

    @pl.when(g + 2 < _TRAJ_CHUNKS)
    def _():
      wait_idx(b2)
      start_gathers(b2)
    wait_gathers(b0)

    @pl.when(g + 3 < _TRAJ_CHUNKS)
    def _():
      start_idx(g + 3, b0)
    start_out(g, b0)

  wait_out((_TRAJ_CHUNKS - 1) % 3)


def _user_body(user_hbm, utab_hbm, uout_hbm, idx_v, rows_v, gsem):
  wid = lax.axis_index("s") * _NC + lax.axis_index("c")
  ubase = wid * _UCH
  pltpu.sync_copy(user_hbm.at[pl.ds(ubase, _UCH)], idx_v)
  copies = [
      pltpu.async_copy(utab_hbm.at[idx_v.at[pl.ds(j * _G, _G)]],
                       rows_v.at[pl.ds(j * _G, _G)], gsem)
      for j in range(_UCH // _G)
  ]
  for c in copies:
    c.wait()
  pltpu.sync_copy(rows_v,
                  uout_hbm.at[pl.ds(ubase, _UCH), pl.ds(0, _D)])


@jax.jit
def _sc_embed(user1d, traj1d, user_table, loc_table):
  mesh = plsc.VectorSubcoreMesh(core_axis_name="c", subcore_axis_name="s")
  traj_fn = pl.kernel(
      _traj_body,
      out_type=jax.ShapeDtypeStruct((_N_TRAJ, 2 * _D), jnp.float32),
      mesh=mesh,
      compiler_params=pltpu.CompilerParams(use_tc_tiling_on_sc=False),
      scratch_types=[
          pltpu.VMEM((3, _CH), jnp.int32),
          pltpu.VMEM((3, _CH, _D), jnp.float32),
          pltpu.SemaphoreType.DMA((3,)),
          pltpu.SemaphoreType.DMA((3,)),
          pltpu.SemaphoreType.DMA((3,)),
      ],
  )
  user_fn = pl.kernel(
      _user_body,
      out_type=jax.ShapeDtypeStruct((_N_USER, 2 * _D), jnp.float32),
      mesh=mesh,
      compiler_params=pltpu.CompilerParams(use_tc_tiling_on_sc=False),
      scratch_types=[
          pltpu.VMEM((_UCH,), jnp.int32),
          pltpu.VMEM((_UCH, _D), jnp.float32),
          pltpu.SemaphoreType.DMA,
      ],
  )
  tout = traj_fn(traj1d, loc_table)
  uout = user_fn(user1d, user_table)
  return uout, tout


def kernel(user, traj, user_table, loc_table):
  user1d = user.astype(jnp.int32)
  traj1d = traj.reshape(-1)
  uout128, tout128 = _sc_embed(user1d, traj1d, user_table, loc_table)
  return (uout128[:, :_D],
          tout128[:, :_D].reshape(16384, 200, _D))


# speedup vs baseline: 1.6850x; 1.0016x over previous
import jax
import jax.numpy as jnp
from jax import lax
from jax.experimental import pallas as pl
from jax.experimental.pallas import tpu as pltpu
from jax.experimental.pallas import tpu_sc as plsc

_INFO = plsc.get_sparse_core_info()
_NC, _NS = _INFO.num_cores, _INFO.num_subcores
_NW = _NC * _NS

_G = 128
_CH = 512
_D = 64

_N_TRAJ = 16384 * 200
_N_USER = 16384
_TRAJ_CHUNKS = _N_TRAJ // (_NW * _CH)
_UCH = _N_USER // _NW


def _traj_body(traj_hbm, ltab_hbm, tout_hbm, idx_v, rows_v, isem, gsem, osem):
  wid = lax.axis_index("s") * _NC + lax.axis_index("c")
  base = wid * _TRAJ_CHUNKS

  def start_idx(g, b):
    pltpu.async_copy(traj_hbm.at[pl.ds((base + g) * _CH, _CH)],
                     idx_v.at[b], isem.at[b])

  def wait_idx(b):
    pltpu.make_async_copy(traj_hbm.at[pl.ds(0, _CH)], idx_v.at[b],
                          isem.at[b]).wait()

  def start_gathers(b):
    pltpu.async_copy(ltab_hbm.at[idx_v.at[b]], rows_v.at[b], gsem.at[b])

  def wait_gathers(b):
    pltpu.make_async_copy(ltab_hbm.at[pl.ds(0, _CH)], rows_v.at[b],
                          gsem.at[b]).wait()

  def start_out(g, b):
    pltpu.async_copy(rows_v.at[b],
                     tout_hbm.at[pl.ds((base + g) * _CH, _CH), pl.ds(0, _D)],
                     osem.at[b])

  def wait_out(b):
    pltpu.make_async_copy(rows_v.at[b],
                          tout_hbm.at[pl.ds(0, _CH), pl.ds(0, _D)],
                          osem.at[b]).wait()

  start_idx(0, 0)
  start_idx(1, 1)
  start_idx(2, 2)
  wait_idx(0)
  start_gathers(0)
  wait_idx(1)
  start_gathers(1)

  @pl.loop(0, _TRAJ_CHUNKS)
  def _(g):
    b0 = lax.rem(g, 3)
    b2 = lax.rem(g + 2, 3)

    @pl.when(g >= 1)
    def _():
      wait_out(b2)

    @pl.when(g + 2 < _TRAJ_CHUNKS)
    def _():
      wait_idx(b2)
      start_gathers(b2)
    wait_gathers(b0)

    @pl.when(g + 3 < _TRAJ_CHUNKS)
    def _():
      start_idx(g + 3, b0)
    start_out(g, b0)

  wait_out((_TRAJ_CHUNKS - 1) % 3)


def _user_body(user_hbm, utab_hbm, uout_hbm, idx_v, rows_v, gsem):
  wid = lax.axis_index("s") * _NC + lax.axis_index("c")
  ubase = wid * _UCH
  pltpu.sync_copy(user_hbm.at[pl.ds(ubase, _UCH)], idx_v)
  copies = [
      pltpu.async_copy(utab_hbm.at[idx_v.at[pl.ds(j * _G, _G)]],
                       rows_v.at[pl.ds(j * _G, _G)], gsem)
      for j in range(_UCH // _G)
  ]
  for c in copies:
    c.wait()
  pltpu.sync_copy(rows_v,
                  uout_hbm.at[pl.ds(ubase, _UCH), pl.ds(0, _D)])


@jax.jit
def _sc_embed(user1d, traj1d, user_table, loc_table):
  mesh = plsc.VectorSubcoreMesh(core_axis_name="c", subcore_axis_name="s")
  traj_fn = pl.kernel(
      _traj_body,
      out_type=jax.ShapeDtypeStruct((_N_TRAJ, 2 * _D), jnp.float32),
      mesh=mesh,
      compiler_params=pltpu.CompilerParams(use_tc_tiling_on_sc=False),
      scratch_types=[
          pltpu.VMEM((3, _CH), jnp.int32),
          pltpu.VMEM((3, _CH, _D), jnp.float32),
          pltpu.SemaphoreType.DMA((3,)),
          pltpu.SemaphoreType.DMA((3,)),
          pltpu.SemaphoreType.DMA((3,)),
      ],
  )
  user_fn = pl.kernel(
      _user_body,
      out_type=jax.ShapeDtypeStruct((_N_USER, 2 * _D), jnp.float32),
      mesh=mesh,
      compiler_params=pltpu.CompilerParams(use_tc_tiling_on_sc=False),
      scratch_types=[
          pltpu.VMEM((_UCH,), jnp.int32),
          pltpu.VMEM((_UCH, _D), jnp.float32),
          pltpu.SemaphoreType.DMA,
      ],
  )
  tout = traj_fn(traj1d, loc_table)
  uout = user_fn(user1d, user_table)
  return uout, tout


def kernel(user, traj, user_table, loc_table):
  user1d = user.astype(jnp.int32)
  traj1d = traj.reshape(-1)
  uout128, tout128 = _sc_embed(user1d, traj1d, user_table, loc_table)
  return (uout128[:, :_D],
          tout128[:, :_D].reshape(16384, 200, _D))
